# baseline (speedup 1.0000x reference)
import functools

import jax
import jax.numpy as jnp
from jax import lax
from jax.experimental import pallas as pl
from jax.experimental.pallas import tpu as pltpu
from jax.experimental.pallas import tpu_sc as plsc

BATCH = 16384
FEAT_DIM = 64
LANES = 16
NUM_ROWS = 1000000

_info = plsc.get_sparse_core_info()
NUM_CORES = _info.num_cores
NUM_SUBCORES = _info.num_subcores
NUM_WORKERS = NUM_CORES * NUM_SUBCORES
B_PER_W = BATCH // NUM_WORKERS
CHUNK = 128


def _sc_body(feat_hbm, pair_hbm, parity_hbm, table_hbm, out_hbm,
             idx_v, par_v, rows_v, feat_v, out_v, sem):
    wid = lax.axis_index("s") * NUM_CORES + lax.axis_index("c")
    base = wid * B_PER_W

    pltpu.sync_copy(pair_hbm.at[pl.ds(base, B_PER_W)], idx_v)
    pltpu.sync_copy(parity_hbm.at[pl.ds(base, B_PER_W)], par_v)

    lane = lax.iota(jnp.int32, LANES)

    def chunk_body(c, carry):
        cbase = c * CHUNK
        gather = pltpu.async_copy(
            table_hbm.at[idx_v.at[pl.ds(cbase, CHUNK)]], rows_v, sem)
        pltpu.sync_copy(feat_hbm.at[pl.ds(base + cbase, CHUNK)], feat_v)
        gather.wait()

        def group_body(g, carry2):
            outvec = jnp.zeros((LANES,), jnp.float32)
            parv = par_v[pl.ds(cbase + g * LANES, LANES)]
            for j in range(LANES):
                r = g * LANES + j
                hi = parv[j] != 0
                acc = jnp.zeros((LANES,), jnp.float32)
                for q in range(FEAT_DIM // LANES):
                    f = feat_v[r, pl.ds(q * LANES, LANES)]
                    wlo = rows_v[r, pl.ds(q * LANES, LANES)]
                    whi = rows_v[r, pl.ds(FEAT_DIM + q * LANES, LANES)]
                    acc = acc + f * jnp.where(hi, whi, wlo)
                total = jnp.sum(acc)
                outvec = jnp.where(lane == j, total, outvec)
            out_v[pl.ds(cbase + g * LANES, LANES)] = outvec
            return carry2

        lax.fori_loop(0, CHUNK // LANES, group_body, 0)
        return carry

    lax.fori_loop(0, B_PER_W // CHUNK, chunk_body, 0)

    pltpu.sync_copy(out_v, out_hbm.at[pl.ds(base, B_PER_W)])


@jax.jit
def _projection_head(feat, pair, parity, table2):
    mesh = plsc.VectorSubcoreMesh(core_axis_name="c", subcore_axis_name="s")
    kern = functools.partial(
        pl.kernel,
        out_type=jax.ShapeDtypeStruct((BATCH,), jnp.float32),
        mesh=mesh,
        scratch_types=[
            pltpu.VMEM((B_PER_W,), jnp.int32),
            pltpu.VMEM((B_PER_W,), jnp.int32),
            pltpu.VMEM((CHUNK, 2 * FEAT_DIM), jnp.float32),
            pltpu.VMEM((CHUNK, FEAT_DIM), jnp.float32),
            pltpu.VMEM((B_PER_W,), jnp.float32),
            pltpu.SemaphoreType.DMA,
        ],
        compiler_params=pltpu.CompilerParams(needs_layout_passes=False),
    )(_sc_body)
    return kern(feat, pair, parity, table2)


def kernel(feat, y, embed_weight):
    y32 = y.astype(jnp.int32)
    pair = y32 >> 1
    parity = y32 & 1
    table2 = embed_weight.reshape(NUM_ROWS // 2, 2 * FEAT_DIM)
    return _projection_head(feat, pair, parity, table2)


# speedup vs baseline: 1.6714x; 1.6714x over previous
import functools

import jax
import jax.numpy as jnp
from jax import lax
from jax.experimental import pallas as pl
from jax.experimental.pallas import tpu as pltpu
from jax.experimental.pallas import tpu_sc as plsc

BATCH = 16384
FEAT_DIM = 64
LANES = 16

_info = plsc.get_sparse_core_info()
NUM_CORES = _info.num_cores
NUM_SUBCORES = _info.num_subcores
NUM_WORKERS = NUM_CORES * NUM_SUBCORES
B_PER_W = BATCH // NUM_WORKERS
CHUNK = 64


def _sc_body(feat_hbm, y_hbm, table_hbm, out_hbm,
             y_v, rows_v, feat_v, out_v, sem):
    wid = lax.axis_index("s") * NUM_CORES + lax.axis_index("c")
    base = wid * B_PER_W

    pltpu.sync_copy(y_hbm.at[pl.ds(base, B_PER_W)], y_v)

    lane = lax.iota(jnp.int32, LANES)

    def chunk_body(c, carry):
        cbase = c * CHUNK
        copies = []
        for g in range(CHUNK // LANES):
            yv = y_v[pl.ds(cbase + g * LANES, LANES)]
            for k in range(LANES):
                r = yv[k]
                copies.append(pltpu.async_copy(
                    table_hbm.at[r], rows_v.at[g * LANES + k], sem))
        pltpu.sync_copy(feat_hbm.at[pl.ds(base + cbase, CHUNK)], feat_v)
        for cp in copies:
            cp.wait()

        def compute_group(g):
            outvec = jnp.zeros((LANES,), jnp.float32)
            for j in range(LANES):
                rr = g * LANES + j
                acc = jnp.zeros((LANES,), jnp.float32)
                for q in range(FEAT_DIM // LANES):
                    f = feat_v[rr, pl.ds(q * LANES, LANES)]
                    w = rows_v[rr, pl.ds(q * LANES, LANES)]
                    acc = acc + f * w
                total = jnp.sum(acc)
                outvec = jnp.where(lane == j, total, outvec)
            out_v[pl.ds(cbase + g * LANES, LANES)] = outvec

        for g in range(CHUNK // LANES):
            compute_group(g)
        return carry

    lax.fori_loop(0, B_PER_W // CHUNK, chunk_body, 0)

    pltpu.sync_copy(out_v, out_hbm.at[pl.ds(base, B_PER_W)])


@jax.jit
def _projection_head(feat, y32, table):
    mesh = plsc.VectorSubcoreMesh(core_axis_name="c", subcore_axis_name="s")
    kern = functools.partial(
        pl.kernel,
        out_type=jax.ShapeDtypeStruct((BATCH,), jnp.float32),
        mesh=mesh,
        scratch_types=[
            pltpu.VMEM((B_PER_W,), jnp.int32),
            pltpu.VMEM((CHUNK, FEAT_DIM), jnp.float32),
            pltpu.VMEM((CHUNK, FEAT_DIM), jnp.float32),
            pltpu.VMEM((B_PER_W,), jnp.float32),
            pltpu.SemaphoreType.DMA,
        ],
        compiler_params=pltpu.CompilerParams(needs_layout_passes=False),
    )(_sc_body)
    return kern(feat, y32, table)


def kernel(feat, y, embed_weight):
    return _projection_head(feat, y.astype(jnp.int32), embed_weight)
